# Initial kernel scaffold; baseline (speedup 1.0000x reference)
import jax
import jax.numpy as jnp
from jax.experimental import pallas as pl


def kernel(x, edge_index, batch, W1l, b1l, W1r, g1, be1, W2l, b2l, W2r, g2, be2, W3l, b3l, W3r, g3, be3, Wn, bnb, W0, b0, Wout, bout):
    raise NotImplementedError("write your pallas kernel here")


# speedup vs baseline: 2.7695x; 2.7695x over previous
import functools

import jax
import jax.numpy as jnp
from jax import lax
from jax.experimental import pallas as pl
from jax.experimental.pallas import tpu as pltpu
from jax.experimental.pallas import tpu_sc as plsc

N = 10000
E = 160000
H = 256
HH = 128
G = 64
NC = 2
NS = 16
CHUNK = 128
EPT = E // NS
NCHUNK = 80
EPT_PAD = NCHUNK * CHUNK
ACC_ROWS = 10112
ROWS_PER_TILE = ACC_ROWS // NS
WB_TILES = 10
WB_ROWS = N // WB_TILES
BR = 1000
NB = N // BR


def _sc_mesh():
  return plsc.VectorSubcoreMesh(core_axis_name="c", subcore_axis_name="s")


def _segsum_body(hl_hbm, srcp_hbm, dstp_hbm, zeros_hbm, out_hbm,
                 src_v, dst_v, buf_v, acc, sem0, sem1):
  c = lax.axis_index("c")
  s = lax.axis_index("s")
  pltpu.sync_copy(zeros_hbm, acc.at[pl.ds(s * ROWS_PER_TILE, ROWS_PER_TILE)])
  w = c * NS + s
  pltpu.sync_copy(srcp_hbm.at[w], src_v)
  pltpu.sync_copy(dstp_hbm.at[s], dst_v)
  plsc.subcore_barrier()
  del sem1
  for j in range(NCHUNK):
    pltpu.async_copy(hl_hbm.at[src_v.at[j]], buf_v, sem0).wait()
    pltpu.sync_copy(buf_v, acc.at[dst_v.at[j]], add=True)
  plsc.subcore_barrier()

  @pl.when(s < WB_TILES)
  def _():
    pltpu.sync_copy(acc.at[pl.ds(s * WB_ROWS, WB_ROWS)],
                    out_hbm.at[c].at[pl.ds(s * WB_ROWS, WB_ROWS)])


def _sc_segsum(hl_flat, srcp, dstp, zeros):
  k = pl.kernel(
      _segsum_body,
      out_type=jax.ShapeDtypeStruct((NC, N, HH), jnp.float32),
      mesh=_sc_mesh(),
      scratch_types=[
          pltpu.VMEM((NCHUNK, CHUNK), jnp.int32),
          pltpu.VMEM((NCHUNK, CHUNK), jnp.int32),
          pltpu.VMEM((CHUNK, HH), jnp.float32),
          pltpu.VMEM_SHARED((ACC_ROWS, HH), jnp.float32),
          pltpu.SemaphoreType.DMA,
          pltpu.SemaphoreType.DMA,
      ],
  )
  return k(hl_flat, srcp, dstp, zeros)


def _count_body(dstp_hbm, ones_hbm, zeros_hbm, out_hbm, dst_v, ones_v, acc):
  s = lax.axis_index("s")
  pltpu.sync_copy(zeros_hbm, acc.at[pl.ds(s * ROWS_PER_TILE, ROWS_PER_TILE)])
  pltpu.sync_copy(dstp_hbm.at[s], dst_v)
  pltpu.sync_copy(ones_hbm, ones_v)
  plsc.subcore_barrier()
  for j in range(NCHUNK):
    pltpu.sync_copy(ones_v, acc.at[dst_v.at[j]], add=True)
  plsc.subcore_barrier()

  @pl.when(s < WB_TILES)
  def _():
    pltpu.sync_copy(acc.at[pl.ds(s * WB_ROWS, WB_ROWS)],
                    out_hbm.at[pl.ds(s * WB_ROWS, WB_ROWS)])


def _sc_count(dstp, ones, zeros):
  k = pl.kernel(
      _count_body,
      out_type=jax.ShapeDtypeStruct((N, HH), jnp.float32),
      mesh=_sc_mesh(),
      scratch_types=[
          pltpu.VMEM((NCHUNK, CHUNK), jnp.int32),
          pltpu.VMEM((CHUNK, HH), jnp.float32),
          pltpu.VMEM_SHARED((ACC_ROWS, HH), jnp.float32),
      ],
  )
  return k(dstp, ones, zeros)


def _dotT(a, w):
  return lax.dot_general(a, w, (((1,), (1,)), ((), ())),
                         preferred_element_type=jnp.float32)


def _pre_body(h_ref, wl_ref, wr_ref, bl_ref, hl_ref, hr_ref):
  hb = h_ref[...]
  hl_ref[...] = _dotT(hb, wl_ref[...])[None]
  hr_ref[...] = _dotT(hb, wr_ref[...]) + bl_ref[0]


def _mm_pre(h, Wl, Wr, bl):
  bl2 = bl.reshape(NC, 1, HH)
  return pl.pallas_call(
      _pre_body,
      grid=(NB, NC),
      in_specs=[
          pl.BlockSpec((BR, H), lambda i, c: (i, 0)),
          pl.BlockSpec((HH, H), lambda i, c: (c, 0)),
          pl.BlockSpec((HH, H), lambda i, c: (c, 0)),
          pl.BlockSpec((1, 1, HH), lambda i, c: (c, 0, 0)),
      ],
      out_specs=[
          pl.BlockSpec((1, BR, HH), lambda i, c: (c, i, 0)),
          pl.BlockSpec((BR, HH), lambda i, c: (i, c)),
      ],
      out_shape=[
          jax.ShapeDtypeStruct((NC, N, HH), jnp.float32),
          jax.ShapeDtypeStruct((N, H), jnp.float32),
      ],
  )(h, Wl, Wr, bl2)


def _combine_body(with_next, s_ref, cnt_ref, hr_ref, g_ref, be_ref,
                  *rest):
  if with_next:
    wl_ref, wr_ref, bl_ref = rest[:3]
    hl_ref, hr2_ref = rest[3:5]
    stat_ref = rest[5]
  else:
    h_ref = rest[0]
    stat_ref = rest[1]
  p = pl.program_id(0)
  i = pl.program_id(1)

  def _t():
    t = jnp.concatenate([s_ref[0], s_ref[1]], axis=-1)
    cnt = jnp.maximum(cnt_ref[:, :1], 1.0)
    return t / cnt + hr_ref[...]

  @pl.when((p == 0) & (i == 0))
  def _():
    stat_ref[...] = jnp.zeros_like(stat_ref)

  @pl.when(p == 0)
  def _():
    t = _t()
    stat_ref[0:1, :] += jnp.sum(t, axis=0, keepdims=True)
    stat_ref[1:2, :] += jnp.sum(t * t, axis=0, keepdims=True)

  @pl.when(p == 1)
  def _():
    t = _t()
    mean = stat_ref[0:1, :] * (1.0 / N)
    var = stat_ref[1:2, :] * (1.0 / N) - mean * mean
    hn = (t - mean) * jax.lax.rsqrt(var + 1e-5) * g_ref[...] + be_ref[...]
    hn = jnp.maximum(hn, 0.0)
    if with_next:
      for cc in range(NC):
        hl_ref[cc] = _dotT(hn, wl_ref[cc])
      hr2_ref[...] = _dotT(hn, wr_ref[...]) + bl_ref[...]
    else:
      h_ref[...] = hn


def _combine_mid(s, cnt16, hr, g, be, Wln, Wrn, bln):
  return pl.pallas_call(
      functools.partial(_combine_body, True),
      grid=(2, NB),
      in_specs=[
          pl.BlockSpec((NC, BR, HH), lambda p, i: (0, i, 0)),
          pl.BlockSpec((BR, HH), lambda p, i: (i, 0)),
          pl.BlockSpec((BR, H), lambda p, i: (i, 0)),
          pl.BlockSpec((1, H), lambda p, i: (0, 0)),
          pl.BlockSpec((1, H), lambda p, i: (0, 0)),
          pl.BlockSpec((NC, HH, H), lambda p, i: (0, 0, 0)),
          pl.BlockSpec((H, H), lambda p, i: (0, 0)),
          pl.BlockSpec((1, H), lambda p, i: (0, 0)),
      ],
      out_specs=[
          pl.BlockSpec((NC, BR, HH), lambda p, i: (0, i, 0)),
          pl.BlockSpec((BR, H), lambda p, i: (i, 0)),
      ],
      out_shape=[
          jax.ShapeDtypeStruct((NC, N, HH), jnp.float32),
          jax.ShapeDtypeStruct((N, H), jnp.float32),
      ],
      scratch_shapes=[pltpu.VMEM((8, H), jnp.float32)],
  )(s, cnt16, hr, g.reshape(1, H), be.reshape(1, H),
    Wln.reshape(NC, HH, H), Wrn, bln.reshape(1, H))


def _combine_last(s, cnt16, hr, g, be):
  return pl.pallas_call(
      functools.partial(_combine_body, False),
      grid=(2, NB),
      in_specs=[
          pl.BlockSpec((NC, BR, HH), lambda p, i: (0, i, 0)),
          pl.BlockSpec((BR, HH), lambda p, i: (i, 0)),
          pl.BlockSpec((BR, H), lambda p, i: (i, 0)),
          pl.BlockSpec((1, H), lambda p, i: (0, 0)),
          pl.BlockSpec((1, H), lambda p, i: (0, 0)),
      ],
      out_specs=pl.BlockSpec((BR, H), lambda p, i: (i, 0)),
      out_shape=jax.ShapeDtypeStruct((N, H), jnp.float32),
      scratch_shapes=[pltpu.VMEM((8, H), jnp.float32)],
  )(s, cnt16, hr, g.reshape(1, H), be.reshape(1, H))


def _pool_body(h3_ref, x_ref, bcol_ref, brow_ref, bprev_ref,
               w0_ref, b0_ref, wn_ref, bnb_ref, wa_ref, wb_ref, bo_ref,
               out_ref, hg_ref, rx_ref):
  i = pl.program_id(0)

  @pl.when(i == 0)
  def _():
    hg_ref[...] = jnp.zeros_like(hg_ref)
    rx_ref[...] = jnp.zeros_like(rx_ref)

  @pl.when(i < NB)
  def _():
    t3 = h3_ref[...]
    bcol = bcol_ref[...]
    for g in range(G):
      m = bcol == g
      v = jnp.max(jnp.where(m, t3, 0.0), axis=0, keepdims=True)
      hg_ref[g:g + 1, :] = jnp.maximum(hg_ref[g:g + 1, :], v)
    brow = brow_ref[0]
    isr = (brow != bprev_ref[0]).astype(jnp.float32)
    gids = lax.broadcasted_iota(jnp.int32, (G, BR), 0)
    sel = jnp.where(gids == brow, 1.0, 0.0) * isr
    rx_ref[...] += jnp.dot(sel, x_ref[...], preferred_element_type=jnp.float32)

  @pl.when(i == NB)
  def _():
    h2 = jnp.maximum(_dotT(hg_ref[...], w0_ref[...]) + b0_ref[...], 0.0)
    news = jnp.maximum(_dotT(rx_ref[...], wn_ref[...]) + bnb_ref[...], 0.0)
    z = _dotT(h2, wa_ref[...]) + _dotT(news, wb_ref[...]) + bo_ref[...]
    out_ref[...] = 1.0 / (1.0 + jnp.exp(-z))


def _pool_final(h3, x, bcol, brow, bprev, W0, b0, Wn, bnb, Wout, bout):
  wa = Wout[:, :H]
  wb = Wout[:, H:]
  cap = lambda i: jnp.minimum(i, NB - 1)
  return pl.pallas_call(
      _pool_body,
      grid=(NB + 1,),
      in_specs=[
          pl.BlockSpec((BR, H), lambda i: (cap(i), 0)),
          pl.BlockSpec((BR, H), lambda i: (cap(i), 0)),
          pl.BlockSpec((BR, 1), lambda i: (cap(i), 0)),
          pl.BlockSpec((1, 1, BR), lambda i: (cap(i), 0, 0)),
          pl.BlockSpec((1, 1, BR), lambda i: (cap(i), 0, 0)),
          pl.BlockSpec((H, H), lambda i: (0, 0)),
          pl.BlockSpec((1, H), lambda i: (0, 0)),
          pl.BlockSpec((H, H), lambda i: (0, 0)),
          pl.BlockSpec((1, H), lambda i: (0, 0)),
          pl.BlockSpec((1, H), lambda i: (0, 0)),
          pl.BlockSpec((1, H), lambda i: (0, 0)),
          pl.BlockSpec((1, 1), lambda i: (0, 0)),
      ],
      out_specs=pl.BlockSpec((G, 1), lambda i: (0, 0)),
      out_shape=jax.ShapeDtypeStruct((G, 1), jnp.float32),
      scratch_shapes=[
          pltpu.VMEM((G, H), jnp.float32),
          pltpu.VMEM((G, H), jnp.float32),
      ],
  )(h3, x, bcol, brow, bprev, W0, b0.reshape(1, H), Wn, bnb.reshape(1, H),
    wa, wb, bout.reshape(1, 1))


def kernel(x, edge_index, batch, W1l, b1l, W1r, g1, be1, W2l, b2l, W2r, g2,
           be2, W3l, b3l, W3r, g3, be3, Wn, bnb, W0, b0, Wout, bout):
  src = edge_index[0].astype(jnp.int32)
  dst = edge_index[1].astype(jnp.int32)

  pad = EPT_PAD - EPT
  srcr = jnp.pad(src.reshape(NS, EPT), ((0, 0), (0, pad)))
  dstr = jnp.pad(dst.reshape(NS, EPT), ((0, 0), (0, pad)),
                 constant_values=N)
  dstp = dstr.reshape(NS, NCHUNK, CHUNK)
  srcp = jnp.stack([srcr, srcr + N]).reshape(NC * NS, NCHUNK, CHUNK)

  zeros = jnp.zeros((ROWS_PER_TILE, HH), jnp.float32)
  ones = jnp.ones((CHUNK, HH), jnp.float32)

  cnt16 = _sc_count(dstp, ones, zeros)

  hl1, hr1 = _mm_pre(x, W1l, W1r, b1l)
  s1 = _sc_segsum(hl1.reshape(NC * N, HH), srcp, dstp, zeros)
  hl2, hr2 = _combine_mid(s1, cnt16, hr1, g1, be1, W2l, W2r, b2l)
  s2 = _sc_segsum(hl2.reshape(NC * N, HH), srcp, dstp, zeros)
  hl3, hr3 = _combine_mid(s2, cnt16, hr2, g2, be2, W3l, W3r, b3l)
  s3 = _sc_segsum(hl3.reshape(NC * N, HH), srcp, dstp, zeros)
  h3 = _combine_last(s3, cnt16, hr3, g3, be3)

  batch_i = batch.astype(jnp.int32)
  bcol = batch_i.reshape(N, 1)
  brow = batch_i.reshape(NB, 1, BR)
  bprev = jnp.concatenate(
      [jnp.full((1,), -1, jnp.int32), batch_i[:-1]]).reshape(NB, 1, BR)

  return _pool_final(h3, x, bcol, brow, bprev, W0, b0, Wn, bnb, Wout, bout)


# speedup vs baseline: 3.0403x; 1.0978x over previous
import functools

import jax
import jax.numpy as jnp
from jax import lax
from jax.experimental import pallas as pl
from jax.experimental.pallas import tpu as pltpu
from jax.experimental.pallas import tpu_sc as plsc

N = 10000
E = 160000
H = 256
HH = 128
G = 64
NC = 2
NS = 16
CHUNK = 128
EPT = E // NS
NCHUNK = 80
QCH = 16
EPT_PAD = NCHUNK * CHUNK
ACC_ROWS = 10112
ROWS_PER_TILE = ACC_ROWS // NS
WB_TILES = 10
WB_ROWS = N // WB_TILES
BR = 1000
NB = N // BR


def _sc_mesh():
  return plsc.VectorSubcoreMesh(core_axis_name="c", subcore_axis_name="s")


def _segsum_body(hl_hbm, srcp_hbm, dstp_hbm, zeros_hbm, out_hbm,
                 src_v, dst_v, buf_v, acc, sem0, sem1):
  c = lax.axis_index("c")
  s = lax.axis_index("s")
  pltpu.sync_copy(zeros_hbm, acc.at[pl.ds(s * ROWS_PER_TILE, ROWS_PER_TILE)])
  w = c * NS + s
  plsc.subcore_barrier()
  sems = (sem0, sem1)
  for q in range(NCHUNK // QCH):
    pltpu.sync_copy(srcp_hbm.at[w].at[pl.ds(q * QCH, QCH)], src_v)
    pltpu.sync_copy(dstp_hbm.at[s].at[pl.ds(q * QCH, QCH)], dst_v)
    cp = [None, None]
    cp[0] = pltpu.async_copy(hl_hbm.at[src_v.at[0]], buf_v.at[0], sems[0])
    for j in range(QCH):
      cp[j % 2].wait()
      if j + 1 < QCH:
        cp[(j + 1) % 2] = pltpu.async_copy(
            hl_hbm.at[src_v.at[j + 1]], buf_v.at[(j + 1) % 2],
            sems[(j + 1) % 2])
      pltpu.sync_copy(buf_v.at[j % 2], acc.at[dst_v.at[j]], add=True)
  plsc.subcore_barrier()

  @pl.when(s < WB_TILES)
  def _():
    pltpu.sync_copy(acc.at[pl.ds(s * WB_ROWS, WB_ROWS)],
                    out_hbm.at[c].at[pl.ds(s * WB_ROWS, WB_ROWS)])


def _sc_segsum(hl_flat, srcp, dstp, zeros):
  k = pl.kernel(
      _segsum_body,
      out_type=jax.ShapeDtypeStruct((NC, N, HH), jnp.float32),
      mesh=_sc_mesh(),
      scratch_types=[
          pltpu.VMEM((QCH, CHUNK), jnp.int32),
          pltpu.VMEM((QCH, CHUNK), jnp.int32),
          pltpu.VMEM((2, CHUNK, HH), jnp.float32),
          pltpu.VMEM_SHARED((ACC_ROWS, HH), jnp.float32),
          pltpu.SemaphoreType.DMA,
          pltpu.SemaphoreType.DMA,
      ],
  )
  return k(hl_flat, srcp, dstp, zeros)


def _count_body(dstp_hbm, ones_hbm, zeros_hbm, out_hbm, dst_v, ones_v, acc):
  s = lax.axis_index("s")
  pltpu.sync_copy(zeros_hbm, acc.at[pl.ds(s * ROWS_PER_TILE, ROWS_PER_TILE)])
  pltpu.sync_copy(dstp_hbm.at[s], dst_v)
  pltpu.sync_copy(ones_hbm, ones_v)
  plsc.subcore_barrier()
  for j in range(NCHUNK):
    pltpu.sync_copy(ones_v, acc.at[dst_v.at[j]], add=True)
  plsc.subcore_barrier()

  @pl.when(s < WB_TILES)
  def _():
    pltpu.sync_copy(acc.at[pl.ds(s * WB_ROWS, WB_ROWS)],
                    out_hbm.at[pl.ds(s * WB_ROWS, WB_ROWS)])


def _sc_count(dstp, ones, zeros):
  k = pl.kernel(
      _count_body,
      out_type=jax.ShapeDtypeStruct((N, HH), jnp.float32),
      mesh=_sc_mesh(),
      scratch_types=[
          pltpu.VMEM((NCHUNK, CHUNK), jnp.int32),
          pltpu.VMEM((CHUNK, HH), jnp.float32),
          pltpu.VMEM_SHARED((ACC_ROWS, HH), jnp.float32),
      ],
  )
  return k(dstp, ones, zeros)


def _dotT(a, w):
  return lax.dot_general(a, w, (((1,), (1,)), ((), ())),
                         preferred_element_type=jnp.float32)


def _pre_body(h_ref, wl_ref, wr_ref, bl_ref, hl_ref, hr_ref):
  hb = h_ref[...]
  hl_ref[...] = _dotT(hb, wl_ref[...])[None]
  hr_ref[...] = _dotT(hb, wr_ref[...]) + bl_ref[0]


def _mm_pre(h, Wl, Wr, bl):
  bl2 = bl.reshape(NC, 1, HH)
  return pl.pallas_call(
      _pre_body,
      grid=(NB, NC),
      in_specs=[
          pl.BlockSpec((BR, H), lambda i, c: (i, 0)),
          pl.BlockSpec((HH, H), lambda i, c: (c, 0)),
          pl.BlockSpec((HH, H), lambda i, c: (c, 0)),
          pl.BlockSpec((1, 1, HH), lambda i, c: (c, 0, 0)),
      ],
      out_specs=[
          pl.BlockSpec((1, BR, HH), lambda i, c: (c, i, 0)),
          pl.BlockSpec((BR, HH), lambda i, c: (i, c)),
      ],
      out_shape=[
          jax.ShapeDtypeStruct((NC, N, HH), jnp.float32),
          jax.ShapeDtypeStruct((N, H), jnp.float32),
      ],
  )(h, Wl, Wr, bl2)


def _combine_body(with_next, s_ref, cnt_ref, hr_ref, g_ref, be_ref,
                  *rest):
  if with_next:
    wl_ref, wr_ref, bl_ref = rest[:3]
    hl_ref, hr2_ref = rest[3:5]
    stat_ref = rest[5]
  else:
    h_ref = rest[0]
    stat_ref = rest[1]
  p = pl.program_id(0)
  i = pl.program_id(1)

  def _t():
    t = jnp.concatenate([s_ref[0], s_ref[1]], axis=-1)
    cnt = jnp.maximum(cnt_ref[:, :1], 1.0)
    return t / cnt + hr_ref[...]

  @pl.when((p == 0) & (i == 0))
  def _():
    stat_ref[...] = jnp.zeros_like(stat_ref)

  @pl.when(p == 0)
  def _():
    t = _t()
    stat_ref[0:1, :] += jnp.sum(t, axis=0, keepdims=True)
    stat_ref[1:2, :] += jnp.sum(t * t, axis=0, keepdims=True)

  @pl.when(p == 1)
  def _():
    t = _t()
    mean = stat_ref[0:1, :] * (1.0 / N)
    var = stat_ref[1:2, :] * (1.0 / N) - mean * mean
    hn = (t - mean) * jax.lax.rsqrt(var + 1e-5) * g_ref[...] + be_ref[...]
    hn = jnp.maximum(hn, 0.0)
    if with_next:
      for cc in range(NC):
        hl_ref[cc] = _dotT(hn, wl_ref[cc])
      hr2_ref[...] = _dotT(hn, wr_ref[...]) + bl_ref[...]
    else:
      h_ref[...] = hn


def _combine_mid(s, cnt16, hr, g, be, Wln, Wrn, bln):
  return pl.pallas_call(
      functools.partial(_combine_body, True),
      grid=(2, NB),
      in_specs=[
          pl.BlockSpec((NC, BR, HH), lambda p, i: (0, i, 0)),
          pl.BlockSpec((BR, HH), lambda p, i: (i, 0)),
          pl.BlockSpec((BR, H), lambda p, i: (i, 0)),
          pl.BlockSpec((1, H), lambda p, i: (0, 0)),
          pl.BlockSpec((1, H), lambda p, i: (0, 0)),
          pl.BlockSpec((NC, HH, H), lambda p, i: (0, 0, 0)),
          pl.BlockSpec((H, H), lambda p, i: (0, 0)),
          pl.BlockSpec((1, H), lambda p, i: (0, 0)),
      ],
      out_specs=[
          pl.BlockSpec((NC, BR, HH), lambda p, i: (0, i, 0)),
          pl.BlockSpec((BR, H), lambda p, i: (i, 0)),
      ],
      out_shape=[
          jax.ShapeDtypeStruct((NC, N, HH), jnp.float32),
          jax.ShapeDtypeStruct((N, H), jnp.float32),
      ],
      scratch_shapes=[pltpu.VMEM((8, H), jnp.float32)],
  )(s, cnt16, hr, g.reshape(1, H), be.reshape(1, H),
    Wln.reshape(NC, HH, H), Wrn, bln.reshape(1, H))


def _combine_last(s, cnt16, hr, g, be):
  return pl.pallas_call(
      functools.partial(_combine_body, False),
      grid=(2, NB),
      in_specs=[
          pl.BlockSpec((NC, BR, HH), lambda p, i: (0, i, 0)),
          pl.BlockSpec((BR, HH), lambda p, i: (i, 0)),
          pl.BlockSpec((BR, H), lambda p, i: (i, 0)),
          pl.BlockSpec((1, H), lambda p, i: (0, 0)),
          pl.BlockSpec((1, H), lambda p, i: (0, 0)),
      ],
      out_specs=pl.BlockSpec((BR, H), lambda p, i: (i, 0)),
      out_shape=jax.ShapeDtypeStruct((N, H), jnp.float32),
      scratch_shapes=[pltpu.VMEM((8, H), jnp.float32)],
  )(s, cnt16, hr, g.reshape(1, H), be.reshape(1, H))


def _pool_body(h3_ref, x_ref, bcol_ref, brow_ref, bprev_ref,
               w0_ref, b0_ref, wn_ref, bnb_ref, wa_ref, wb_ref, bo_ref,
               out_ref, hg_ref, rx_ref):
  i = pl.program_id(0)

  @pl.when(i == 0)
  def _():
    hg_ref[...] = jnp.zeros_like(hg_ref)
    rx_ref[...] = jnp.zeros_like(rx_ref)

  @pl.when(i < NB)
  def _():
    t3 = h3_ref[...]
    bcol = bcol_ref[...]
    for g in range(G):
      m = bcol == g
      v = jnp.max(jnp.where(m, t3, 0.0), axis=0, keepdims=True)
      hg_ref[g:g + 1, :] = jnp.maximum(hg_ref[g:g + 1, :], v)
    brow = brow_ref[0]
    isr = (brow != bprev_ref[0]).astype(jnp.float32)
    gids = lax.broadcasted_iota(jnp.int32, (G, BR), 0)
    sel = jnp.where(gids == brow, 1.0, 0.0) * isr
    rx_ref[...] += jnp.dot(sel, x_ref[...], preferred_element_type=jnp.float32)

  @pl.when(i == NB)
  def _():
    h2 = jnp.maximum(_dotT(hg_ref[...], w0_ref[...]) + b0_ref[...], 0.0)
    news = jnp.maximum(_dotT(rx_ref[...], wn_ref[...]) + bnb_ref[...], 0.0)
    z = _dotT(h2, wa_ref[...]) + _dotT(news, wb_ref[...]) + bo_ref[...]
    out_ref[...] = 1.0 / (1.0 + jnp.exp(-z))


def _pool_final(h3, x, bcol, brow, bprev, W0, b0, Wn, bnb, Wout, bout):
  wa = Wout[:, :H]
  wb = Wout[:, H:]
  cap = lambda i: jnp.minimum(i, NB - 1)
  return pl.pallas_call(
      _pool_body,
      grid=(NB + 1,),
      in_specs=[
          pl.BlockSpec((BR, H), lambda i: (cap(i), 0)),
          pl.BlockSpec((BR, H), lambda i: (cap(i), 0)),
          pl.BlockSpec((BR, 1), lambda i: (cap(i), 0)),
          pl.BlockSpec((1, 1, BR), lambda i: (cap(i), 0, 0)),
          pl.BlockSpec((1, 1, BR), lambda i: (cap(i), 0, 0)),
          pl.BlockSpec((H, H), lambda i: (0, 0)),
          pl.BlockSpec((1, H), lambda i: (0, 0)),
          pl.BlockSpec((H, H), lambda i: (0, 0)),
          pl.BlockSpec((1, H), lambda i: (0, 0)),
          pl.BlockSpec((1, H), lambda i: (0, 0)),
          pl.BlockSpec((1, H), lambda i: (0, 0)),
          pl.BlockSpec((1, 1), lambda i: (0, 0)),
      ],
      out_specs=pl.BlockSpec((G, 1), lambda i: (0, 0)),
      out_shape=jax.ShapeDtypeStruct((G, 1), jnp.float32),
      scratch_shapes=[
          pltpu.VMEM((G, H), jnp.float32),
          pltpu.VMEM((G, H), jnp.float32),
      ],
  )(h3, x, bcol, brow, bprev, W0, b0.reshape(1, H), Wn, bnb.reshape(1, H),
    wa, wb, bout.reshape(1, 1))


def kernel(x, edge_index, batch, W1l, b1l, W1r, g1, be1, W2l, b2l, W2r, g2,
           be2, W3l, b3l, W3r, g3, be3, Wn, bnb, W0, b0, Wout, bout):
  src = edge_index[0].astype(jnp.int32)
  dst = edge_index[1].astype(jnp.int32)

  pad = EPT_PAD - EPT
  srcr = jnp.pad(src.reshape(NS, EPT), ((0, 0), (0, pad)))
  dstr = jnp.pad(dst.reshape(NS, EPT), ((0, 0), (0, pad)),
                 constant_values=N)
  dstp = dstr.reshape(NS, NCHUNK, CHUNK)
  srcp = jnp.stack([srcr, srcr + N]).reshape(NC * NS, NCHUNK, CHUNK)

  zeros = jnp.zeros((ROWS_PER_TILE, HH), jnp.float32)
  ones = jnp.ones((CHUNK, HH), jnp.float32)

  cnt16 = _sc_count(dstp, ones, zeros)

  hl1, hr1 = _mm_pre(x, W1l, W1r, b1l)
  s1 = _sc_segsum(hl1.reshape(NC * N, HH), srcp, dstp, zeros)
  hl2, hr2 = _combine_mid(s1, cnt16, hr1, g1, be1, W2l, W2r, b2l)
  s2 = _sc_segsum(hl2.reshape(NC * N, HH), srcp, dstp, zeros)
  hl3, hr3 = _combine_mid(s2, cnt16, hr2, g2, be2, W3l, W3r, b3l)
  s3 = _sc_segsum(hl3.reshape(NC * N, HH), srcp, dstp, zeros)
  h3 = _combine_last(s3, cnt16, hr3, g3, be3)

  batch_i = batch.astype(jnp.int32)
  bcol = batch_i.reshape(N, 1)
  brow = batch_i.reshape(NB, 1, BR)
  bprev = jnp.concatenate(
      [jnp.full((1,), -1, jnp.int32), batch_i[:-1]]).reshape(NB, 1, BR)

  return _pool_final(h3, x, bcol, brow, bprev, W0, b0, Wn, bnb, Wout, bout)


# speedup vs baseline: 3.1142x; 1.0243x over previous
import functools

import jax
import jax.numpy as jnp
from jax import lax
from jax.experimental import pallas as pl
from jax.experimental.pallas import tpu as pltpu
from jax.experimental.pallas import tpu_sc as plsc

N = 10000
E = 160000
H = 256
HH = 128
G = 64
NC = 2
NS = 16
CHUNK = 128
EPT = E // NS
NCHUNK = 80
QCH = 16
EPT_PAD = NCHUNK * CHUNK
ACC_ROWS = 10112
ROWS_PER_TILE = ACC_ROWS // NS
WB_TILES = 10
WB_ROWS = N // WB_TILES
BR = 1000
NB = N // BR


def _sc_mesh():
  return plsc.VectorSubcoreMesh(core_axis_name="c", subcore_axis_name="s")


def _segsum_body(hl_hbm, srcp_hbm, dstp_hbm, zeros_hbm, out_hbm,
                 src_v, dst_v, buf_v, acc, sem0, sem1):
  c = lax.axis_index("c")
  s = lax.axis_index("s")
  pltpu.sync_copy(zeros_hbm, acc.at[pl.ds(s * ROWS_PER_TILE, ROWS_PER_TILE)])
  w = c * NS + s
  plsc.subcore_barrier()
  sems = (sem0, sem1)
  for q in range(NCHUNK // QCH):
    pltpu.sync_copy(srcp_hbm.at[w].at[pl.ds(q * QCH, QCH)], src_v)
    pltpu.sync_copy(dstp_hbm.at[s].at[pl.ds(q * QCH, QCH)], dst_v)
    cp = [None, None]
    cp[0] = pltpu.async_copy(hl_hbm.at[src_v.at[0]], buf_v.at[0], sems[0])
    for j in range(QCH):
      cp[j % 2].wait()
      if j + 1 < QCH:
        cp[(j + 1) % 2] = pltpu.async_copy(
            hl_hbm.at[src_v.at[j + 1]], buf_v.at[(j + 1) % 2],
            sems[(j + 1) % 2])
      pltpu.sync_copy(buf_v.at[j % 2], acc.at[dst_v.at[j]], add=True)
  plsc.subcore_barrier()

  @pl.when(s < WB_TILES)
  def _():
    pltpu.sync_copy(acc.at[pl.ds(s * WB_ROWS, WB_ROWS)],
                    out_hbm.at[c].at[pl.ds(s * WB_ROWS, WB_ROWS)])


def _sc_segsum(hl_flat, srcp, dstp, zeros):
  k = pl.kernel(
      _segsum_body,
      out_type=jax.ShapeDtypeStruct((NC, N, HH), jnp.float32),
      mesh=_sc_mesh(),
      scratch_types=[
          pltpu.VMEM((QCH, CHUNK), jnp.int32),
          pltpu.VMEM((QCH, CHUNK), jnp.int32),
          pltpu.VMEM((2, CHUNK, HH), jnp.float32),
          pltpu.VMEM_SHARED((ACC_ROWS, HH), jnp.float32),
          pltpu.SemaphoreType.DMA,
          pltpu.SemaphoreType.DMA,
      ],
  )
  return k(hl_flat, srcp, dstp, zeros)


def _count_body(dstp_hbm, ones_hbm, zeros_hbm, out_hbm, dst_v, ones_v, acc):
  s = lax.axis_index("s")
  pltpu.sync_copy(zeros_hbm, acc.at[pl.ds(s * ROWS_PER_TILE, ROWS_PER_TILE)])
  pltpu.sync_copy(dstp_hbm.at[s], dst_v)
  pltpu.sync_copy(ones_hbm, ones_v)
  plsc.subcore_barrier()
  for j in range(NCHUNK):
    pltpu.sync_copy(ones_v, acc.at[dst_v.at[j]], add=True)
  plsc.subcore_barrier()

  @pl.when(s < WB_TILES)
  def _():
    pltpu.sync_copy(acc.at[pl.ds(s * WB_ROWS, WB_ROWS)],
                    out_hbm.at[pl.ds(s * WB_ROWS, WB_ROWS)])


def _sc_count(dstp, ones, zeros):
  k = pl.kernel(
      _count_body,
      out_type=jax.ShapeDtypeStruct((N, HH), jnp.float32),
      mesh=_sc_mesh(),
      scratch_types=[
          pltpu.VMEM((NCHUNK, CHUNK), jnp.int32),
          pltpu.VMEM((CHUNK, HH), jnp.float32),
          pltpu.VMEM_SHARED((ACC_ROWS, HH), jnp.float32),
      ],
  )
  return k(dstp, ones, zeros)


def _dotT(a, w):
  return lax.dot_general(a, w, (((1,), (1,)), ((), ())),
                         preferred_element_type=jnp.float32)


def _pre_body(h_ref, wl_ref, wr_ref, bl_ref, hl_ref, hr_ref):
  hb = h_ref[...]
  hl_ref[...] = _dotT(hb, wl_ref[...])[None]
  hr_ref[...] = _dotT(hb, wr_ref[...]) + bl_ref[0]


def _mm_pre(h, Wl, Wr, bl):
  bl2 = bl.reshape(NC, 1, HH)
  return pl.pallas_call(
      _pre_body,
      grid=(NB, NC),
      in_specs=[
          pl.BlockSpec((BR, H), lambda i, c: (i, 0)),
          pl.BlockSpec((HH, H), lambda i, c: (c, 0)),
          pl.BlockSpec((HH, H), lambda i, c: (c, 0)),
          pl.BlockSpec((1, 1, HH), lambda i, c: (c, 0, 0)),
      ],
      out_specs=[
          pl.BlockSpec((1, BR, HH), lambda i, c: (c, i, 0)),
          pl.BlockSpec((BR, HH), lambda i, c: (i, c)),
      ],
      out_shape=[
          jax.ShapeDtypeStruct((NC, N, HH), jnp.float32),
          jax.ShapeDtypeStruct((N, H), jnp.float32),
      ],
  )(h, Wl, Wr, bl2)


def _combine_body(with_next, s_ref, cnt_ref, hr_ref, g_ref, be_ref,
                  *rest):
  if with_next:
    wl_ref, wr_ref, bl_ref = rest[:3]
    hl_ref, hr2_ref = rest[3:5]
    stat_ref, t_ref = rest[5:7]
  else:
    h_ref = rest[0]
    stat_ref, t_ref = rest[1:3]
  p = pl.program_id(0)
  i = pl.program_id(1)

  @pl.when((p == 0) & (i == 0))
  def _():
    stat_ref[...] = jnp.zeros_like(stat_ref)

  @pl.when(p == 0)
  def _():
    t = jnp.concatenate([s_ref[0], s_ref[1]], axis=-1)
    cnt = jnp.maximum(cnt_ref[:, :1], 1.0)
    t = t / cnt + hr_ref[...]
    t_ref[i] = t
    stat_ref[0:1, :] += jnp.sum(t, axis=0, keepdims=True)
    stat_ref[1:2, :] += jnp.sum(t * t, axis=0, keepdims=True)

  @pl.when(p == 1)
  def _():
    t = t_ref[i]
    mean = stat_ref[0:1, :] * (1.0 / N)
    var = stat_ref[1:2, :] * (1.0 / N) - mean * mean
    hn = (t - mean) * jax.lax.rsqrt(var + 1e-5) * g_ref[...] + be_ref[...]
    hn = jnp.maximum(hn, 0.0)
    if with_next:
      for cc in range(NC):
        hl_ref[cc] = _dotT(hn, wl_ref[cc])
      hr2_ref[...] = _dotT(hn, wr_ref[...]) + bl_ref[...]
    else:
      h_ref[...] = hn


def _p0_map(p, i):
  return jnp.where(p == 0, i, 0)


def _p1_map(p, i):
  return jnp.where(p == 1, i, 0)


def _combine_mid(s, cnt16, hr, g, be, Wln, Wrn, bln):
  return pl.pallas_call(
      functools.partial(_combine_body, True),
      grid=(2, NB),
      in_specs=[
          pl.BlockSpec((NC, BR, HH), lambda p, i: (0, _p0_map(p, i), 0)),
          pl.BlockSpec((BR, HH), lambda p, i: (_p0_map(p, i), 0)),
          pl.BlockSpec((BR, H), lambda p, i: (_p0_map(p, i), 0)),
          pl.BlockSpec((1, H), lambda p, i: (0, 0)),
          pl.BlockSpec((1, H), lambda p, i: (0, 0)),
          pl.BlockSpec((NC, HH, H), lambda p, i: (0, 0, 0)),
          pl.BlockSpec((H, H), lambda p, i: (0, 0)),
          pl.BlockSpec((1, H), lambda p, i: (0, 0)),
      ],
      out_specs=[
          pl.BlockSpec((NC, BR, HH), lambda p, i: (0, _p1_map(p, i), 0)),
          pl.BlockSpec((BR, H), lambda p, i: (_p1_map(p, i), 0)),
      ],
      out_shape=[
          jax.ShapeDtypeStruct((NC, N, HH), jnp.float32),
          jax.ShapeDtypeStruct((N, H), jnp.float32),
      ],
      scratch_shapes=[
          pltpu.VMEM((8, H), jnp.float32),
          pltpu.VMEM((NB, BR, H), jnp.float32),
      ],
  )(s, cnt16, hr, g.reshape(1, H), be.reshape(1, H),
    Wln.reshape(NC, HH, H), Wrn, bln.reshape(1, H))


def _combine_last(s, cnt16, hr, g, be):
  return pl.pallas_call(
      functools.partial(_combine_body, False),
      grid=(2, NB),
      in_specs=[
          pl.BlockSpec((NC, BR, HH), lambda p, i: (0, _p0_map(p, i), 0)),
          pl.BlockSpec((BR, HH), lambda p, i: (_p0_map(p, i), 0)),
          pl.BlockSpec((BR, H), lambda p, i: (_p0_map(p, i), 0)),
          pl.BlockSpec((1, H), lambda p, i: (0, 0)),
          pl.BlockSpec((1, H), lambda p, i: (0, 0)),
      ],
      out_specs=pl.BlockSpec((BR, H), lambda p, i: (_p1_map(p, i), 0)),
      out_shape=jax.ShapeDtypeStruct((N, H), jnp.float32),
      scratch_shapes=[
          pltpu.VMEM((8, H), jnp.float32),
          pltpu.VMEM((NB, BR, H), jnp.float32),
      ],
  )(s, cnt16, hr, g.reshape(1, H), be.reshape(1, H))


def _pool_body(h3_ref, x_ref, bcol_ref, brow_ref, bprev_ref,
               w0_ref, b0_ref, wn_ref, bnb_ref, wa_ref, wb_ref, bo_ref,
               out_ref, hg_ref, rx_ref):
  i = pl.program_id(0)

  @pl.when(i == 0)
  def _():
    hg_ref[...] = jnp.zeros_like(hg_ref)
    rx_ref[...] = jnp.zeros_like(rx_ref)

  @pl.when(i < NB)
  def _():
    t3 = h3_ref[...]
    bcol = bcol_ref[...]
    for g in range(G):
      m = bcol == g
      v = jnp.max(jnp.where(m, t3, 0.0), axis=0, keepdims=True)
      hg_ref[g:g + 1, :] = jnp.maximum(hg_ref[g:g + 1, :], v)
    brow = brow_ref[0]
    isr = (brow != bprev_ref[0]).astype(jnp.float32)
    gids = lax.broadcasted_iota(jnp.int32, (G, BR), 0)
    sel = jnp.where(gids == brow, 1.0, 0.0) * isr
    rx_ref[...] += jnp.dot(sel, x_ref[...], preferred_element_type=jnp.float32)

  @pl.when(i == NB)
  def _():
    h2 = jnp.maximum(_dotT(hg_ref[...], w0_ref[...]) + b0_ref[...], 0.0)
    news = jnp.maximum(_dotT(rx_ref[...], wn_ref[...]) + bnb_ref[...], 0.0)
    z = _dotT(h2, wa_ref[...]) + _dotT(news, wb_ref[...]) + bo_ref[...]
    out_ref[...] = 1.0 / (1.0 + jnp.exp(-z))


def _pool_final(h3, x, bcol, brow, bprev, W0, b0, Wn, bnb, Wout, bout):
  wa = Wout[:, :H]
  wb = Wout[:, H:]
  cap = lambda i: jnp.minimum(i, NB - 1)
  return pl.pallas_call(
      _pool_body,
      grid=(NB + 1,),
      in_specs=[
          pl.BlockSpec((BR, H), lambda i: (cap(i), 0)),
          pl.BlockSpec((BR, H), lambda i: (cap(i), 0)),
          pl.BlockSpec((BR, 1), lambda i: (cap(i), 0)),
          pl.BlockSpec((1, 1, BR), lambda i: (cap(i), 0, 0)),
          pl.BlockSpec((1, 1, BR), lambda i: (cap(i), 0, 0)),
          pl.BlockSpec((H, H), lambda i: (0, 0)),
          pl.BlockSpec((1, H), lambda i: (0, 0)),
          pl.BlockSpec((H, H), lambda i: (0, 0)),
          pl.BlockSpec((1, H), lambda i: (0, 0)),
          pl.BlockSpec((1, H), lambda i: (0, 0)),
          pl.BlockSpec((1, H), lambda i: (0, 0)),
          pl.BlockSpec((1, 1), lambda i: (0, 0)),
      ],
      out_specs=pl.BlockSpec((G, 1), lambda i: (0, 0)),
      out_shape=jax.ShapeDtypeStruct((G, 1), jnp.float32),
      scratch_shapes=[
          pltpu.VMEM((G, H), jnp.float32),
          pltpu.VMEM((G, H), jnp.float32),
      ],
  )(h3, x, bcol, brow, bprev, W0, b0.reshape(1, H), Wn, bnb.reshape(1, H),
    wa, wb, bout.reshape(1, 1))


def kernel(x, edge_index, batch, W1l, b1l, W1r, g1, be1, W2l, b2l, W2r, g2,
           be2, W3l, b3l, W3r, g3, be3, Wn, bnb, W0, b0, Wout, bout):
  src = edge_index[0].astype(jnp.int32)
  dst = edge_index[1].astype(jnp.int32)

  pad = EPT_PAD - EPT
  srcr = jnp.pad(src.reshape(NS, EPT), ((0, 0), (0, pad)))
  dstr = jnp.pad(dst.reshape(NS, EPT), ((0, 0), (0, pad)),
                 constant_values=N)
  dstp = dstr.reshape(NS, NCHUNK, CHUNK)
  srcp = jnp.stack([srcr, srcr + N]).reshape(NC * NS, NCHUNK, CHUNK)

  zeros = jnp.zeros((ROWS_PER_TILE, HH), jnp.float32)
  ones = jnp.ones((CHUNK, HH), jnp.float32)

  cnt16 = _sc_count(dstp, ones, zeros)

  hl1, hr1 = _mm_pre(x, W1l, W1r, b1l)
  s1 = _sc_segsum(hl1.reshape(NC * N, HH), srcp, dstp, zeros)
  hl2, hr2 = _combine_mid(s1, cnt16, hr1, g1, be1, W2l, W2r, b2l)
  s2 = _sc_segsum(hl2.reshape(NC * N, HH), srcp, dstp, zeros)
  hl3, hr3 = _combine_mid(s2, cnt16, hr2, g2, be2, W3l, W3r, b3l)
  s3 = _sc_segsum(hl3.reshape(NC * N, HH), srcp, dstp, zeros)
  h3 = _combine_last(s3, cnt16, hr3, g3, be3)

  batch_i = batch.astype(jnp.int32)
  bcol = batch_i.reshape(N, 1)
  brow = batch_i.reshape(NB, 1, BR)
  bprev = jnp.concatenate(
      [jnp.full((1,), -1, jnp.int32), batch_i[:-1]]).reshape(NB, 1, BR)

  return _pool_final(h3, x, bcol, brow, bprev, W0, b0, Wn, bnb, Wout, bout)
